# Initial kernel scaffold; baseline (speedup 1.0000x reference)
import jax
import jax.numpy as jnp
from jax.experimental import pallas as pl


def kernel(bg_prob, ed_prob, w1, b1, g1, be1, w2, b2, g2, be2, w3, b3, key_store, store_labels, context_mask, add_mode):
    raise NotImplementedError("write your pallas kernel here")


# speedup vs baseline: 10.9689x; 10.9689x over previous
import functools

import jax
import jax.numpy as jnp
from jax import lax
from jax.experimental import pallas as pl
from jax.experimental.pallas import tpu as pltpu

CD, CH, CW = 16, 32, 32
N_VOX = CD * CH * CW
N_KEYS = 4096
LATENT = 8
TOPK = 10
ALPHA = 10.0
QTILE = 256


def _coords():
    n = lax.broadcasted_iota(jnp.int32, (1, N_VOX), 1)
    xc = n % CW
    yc = (n // CW) % CH
    zc = n // (CW * CH)
    return xc, yc, zc


def _shift(x, s):
    c, n = x.shape
    if s == 0:
        return x
    z = jnp.zeros((c, abs(s)), x.dtype)
    if s > 0:
        return jnp.concatenate([x[:, s:], z], axis=1)
    return jnp.concatenate([z, x[:, : n + s]], axis=1)


def _conv3x3(x, wf, cout, xc, yc, zc):
    acc = None
    t = 0
    for dz in (-1, 0, 1):
        mz = (zc + dz >= 0) & (zc + dz <= CD - 1)
        for dy in (-1, 0, 1):
            my = (yc + dy >= 0) & (yc + dy <= CH - 1)
            for dx in (-1, 0, 1):
                mx = (xc + dx >= 0) & (xc + dx <= CW - 1)
                m = mz & my & mx
                s = dz * (CH * CW) + dy * CW + dx
                xs = jnp.where(m, _shift(x, s), 0.0)
                w = wf[t * cout : (t + 1) * cout, :]
                p = jnp.dot(w, xs, preferred_element_type=jnp.float32)
                acc = p if acc is None else acc + p
                t += 1
    return acc


def _bn_relu(h, g, b):
    m = jnp.mean(h, axis=1, keepdims=True)
    v = jnp.mean((h - m) * (h - m), axis=1, keepdims=True)
    return jnp.maximum((h - m) * lax.rsqrt(v + 1e-5) * g + b, 0.0)


def _enc_body(x_ref, w1_ref, b1_ref, g1_ref, be1_ref, w2_ref, b2_ref, g2_ref,
              be2_ref, w3_ref, b3_ref, q_ref):
    xc, yc, zc = _coords()
    h = _conv3x3(x_ref[...], w1_ref[...], 16, xc, yc, zc) + b1_ref[...]
    h = _bn_relu(h, g1_ref[...], be1_ref[...])
    h = _conv3x3(h, w2_ref[...], 32, xc, yc, zc) + b2_ref[...]
    h = _bn_relu(h, g2_ref[...], be2_ref[...])
    lat = jnp.dot(w3_ref[...], h, preferred_element_type=jnp.float32) + b3_ref[...]
    nrm = jnp.sqrt(jnp.sum(lat * lat, axis=0, keepdims=True))
    lat = lat / jnp.maximum(nrm, 1e-12)
    q_ref[...] = lat.T


def _knn_body(q_ref, kt_ref, lab_ref, out_ref):
    q = q_ref[...]
    kt = kt_ref[...]
    lab = lab_ref[...]
    kn = jnp.sum(kt * kt, axis=0, keepdims=True)
    qn = jnp.sum(q * q, axis=1, keepdims=True)
    d = qn - 2.0 * jnp.dot(q, kt, preferred_element_type=jnp.float32) + kn
    col = lax.broadcasted_iota(jnp.int32, (QTILE, N_KEYS), 1)
    wsum = jnp.zeros((QTILE, 1), jnp.float32)
    wl = jnp.zeros((QTILE, 1), jnp.float32)
    for _ in range(TOPK):
        m = jnp.min(d, axis=1, keepdims=True)
        tied = d == m
        idx = jnp.min(jnp.where(tied, col, jnp.int32(1 << 30)), axis=1,
                      keepdims=True)
        sel = col == idx
        l = jnp.sum(jnp.where(sel, lab, 0.0), axis=1, keepdims=True)
        w = jnp.exp(-ALPHA * m)
        wsum += w
        wl += w * l
        d = jnp.where(sel, jnp.float32(jnp.inf), d)
    out_ref[...] = wl / (wsum + 1e-8)


def kernel(bg_prob, ed_prob, w1, b1, g1, be1, w2, b2, g2, be2, w3, b3,
           key_store, store_labels, context_mask, add_mode):
    x = jnp.concatenate([bg_prob, ed_prob], axis=1).reshape(2, N_VOX)
    w1f = jnp.transpose(w1.reshape(16, 2, 27), (2, 0, 1)).reshape(27 * 16, 2)
    w2f = jnp.transpose(w2.reshape(32, 16, 27), (2, 0, 1)).reshape(27 * 32, 16)
    w3f = w3.reshape(LATENT, 32)
    b1c, g1c, be1c = b1.reshape(16, 1), g1.reshape(16, 1), be1.reshape(16, 1)
    b2c, g2c, be2c = b2.reshape(32, 1), g2.reshape(32, 1), be2.reshape(32, 1)
    b3c = b3.reshape(LATENT, 1)
    kt = key_store.T
    lab = store_labels.reshape(1, N_KEYS)

    q = pl.pallas_call(
        _enc_body,
        out_shape=jax.ShapeDtypeStruct((N_VOX, LATENT), jnp.float32),
    )(x, w1f, b1c, g1c, be1c, w2f, b2c, g2c, be2c, w3f, b3c)

    ntile = N_VOX // QTILE
    prob = pl.pallas_call(
        _knn_body,
        grid=(ntile,),
        in_specs=[
            pl.BlockSpec((QTILE, LATENT), lambda i: (i, 0)),
            pl.BlockSpec((LATENT, N_KEYS), lambda i: (0, 0)),
            pl.BlockSpec((1, N_KEYS), lambda i: (0, 0)),
        ],
        out_specs=pl.BlockSpec((QTILE, 1), lambda i: (i, 0)),
        out_shape=jax.ShapeDtypeStruct((N_VOX, 1), jnp.float32),
    )(q, kt, lab)
    return prob.reshape(1, CD, CH, CW)
